# baseline (speedup 1.0000x reference)
import functools

import jax
import jax.numpy as jnp
from jax import lax
from jax.experimental import pallas as pl
from jax.experimental.pallas import tpu as pltpu
from jax.experimental.pallas import tpu_sc as plsc

_NBINS = 38
_N = 1024
_NBATCH = 2
_NC = 2
_NS = 16
_NW = _NC * _NS
_QROWS = 256
_NQ = _NBATCH * _N * (_N // _QROWS)
_QPW = _NQ // _NW
_GRP = 16
_NGRP = _QPW // _GRP


def _bidx_kernel(cp_ref, ct_ref, aux_ref, out_ref):
    d2 = None
    for c in range(3):
        a = cp_ref[0, :, c : c + 1]
        b = ct_ref[0, c : c + 1, :]
        diff = a - b
        d2 = diff * diff if d2 is None else d2 + diff * diff
    d = jnp.sqrt(d2)
    start = aux_ref[1:2, 0:1]
    inv_step = aux_ref[1:2, 1:2]
    bidx = jnp.clip(jnp.floor((d - start) * inv_step), 0.0, _NBINS - 1.0)
    out_ref[:, :] = bidx.astype(jnp.int32)


_sc_mesh = plsc.VectorSubcoreMesh(core_axis_name="c", subcore_axis_name="s")

_FLAT = _N * _NBINS
_PPW = _NBATCH * _N // _NW
_PGRP = 4
_NPG = _PPW // _PGRP


@functools.partial(
    pl.kernel,
    out_type=jax.ShapeDtypeStruct((_NBATCH * _N, _N, _NBINS), jnp.float32),
    mesh=_sc_mesh,
    scratch_types=[
        pltpu.VMEM((2, 1, _N, _NBINS), jnp.float32),
        pltpu.VMEM((_PGRP, _N), jnp.int32),
        pltpu.SemaphoreType.DMA((2,)),
    ],
    compiler_params=pltpu.CompilerParams(
        needs_layout_passes=False, use_tc_tiling_on_sc=False
    ),
)
def _sc_expand(bidx_hbm, zeros_hbm, out_hbm, planes, idxs, osem):
    cid = lax.axis_index("c")
    sid = lax.axis_index("s")
    wid = sid * _NC + cid
    row0 = wid * _PPW

    iota16 = lax.iota(jnp.int32, 16)
    z16 = jnp.full((16,), 0, jnp.int32)
    ones16 = jnp.full((16,), 1.0, jnp.float32)
    zeros16 = jnp.full((16,), 0.0, jnp.float32)

    pltpu.sync_copy(zeros_hbm, planes.at[0])
    pltpu.sync_copy(zeros_hbm, planes.at[1])

    def scat(slot, row, vals):
        for c in range(_N // 16):
            kv = idxs[row, pl.ds(c * 16, 16)]
            jv = iota16 + c * 16
            plsc.store_scatter(planes.at[slot], [z16, jv, kv], vals)

    def group(grp, _):
        gr0 = row0 + grp * _PGRP

        @pl.when(grp > 0)
        def _():
            for slot in (0, 1):
                pltpu.make_async_copy(
                    planes.at[slot],
                    out_hbm.at[pl.ds(gr0 - 2 + slot, 1)],
                    osem.at[slot],
                ).wait()
                scat(slot, 2 + slot, zeros16)

        pltpu.sync_copy(bidx_hbm.at[pl.ds(gr0, _PGRP)], idxs)

        for u in range(_PGRP):
            slot = u % 2
            if u >= 2:
                pltpu.make_async_copy(
                    planes.at[slot],
                    out_hbm.at[pl.ds(gr0 + u - 2, 1)],
                    osem.at[slot],
                ).wait()
                scat(slot, u - 2, zeros16)
            scat(slot, u, ones16)
            pltpu.make_async_copy(
                planes.at[slot], out_hbm.at[pl.ds(gr0 + u, 1)], osem.at[slot]
            ).start()
        return ()

    lax.fori_loop(0, _NPG, group, ())

    last = row0 + _PPW - 1
    pltpu.make_async_copy(planes.at[0], out_hbm.at[pl.ds(last - 1, 1)], osem.at[0]).wait()
    pltpu.make_async_copy(planes.at[1], out_hbm.at[pl.ds(last, 1)], osem.at[1]).wait()


def kernel(CB_coords, v_bins):
    nbatch, n, _ = CB_coords.shape
    coords_p = jnp.pad(CB_coords, ((0, 0), (0, 0), (0, 5)))
    coords_t = jnp.pad(
        jnp.transpose(CB_coords, (0, 2, 1)), ((0, 0), (0, 5), (0, 0))
    )
    aux = jnp.zeros((8, 128), jnp.float32)
    aux = aux.at[1, 0].set(v_bins[0])
    aux = aux.at[1, 1].set(1.0 / (v_bins[1] - v_bins[0]))

    bidx = pl.pallas_call(
        _bidx_kernel,
        grid=(nbatch,),
        in_specs=[
            pl.BlockSpec((1, n, 8), lambda b: (b, 0, 0)),
            pl.BlockSpec((1, 8, n), lambda b: (b, 0, 0)),
            pl.BlockSpec((8, 128), lambda b: (0, 0)),
        ],
        out_specs=pl.BlockSpec((n, n), lambda b: (b, 0)),
        out_shape=jax.ShapeDtypeStruct((nbatch * n, n), jnp.int32),
    )(coords_p, coords_t, aux)

    zeros = jnp.zeros((1, _N, _NBINS), jnp.float32)
    out = _sc_expand(bidx, zeros)
    return out.reshape(nbatch, n, n, _NBINS)


# speedup vs baseline: 1.7334x; 1.7334x over previous
import jax
import jax.numpy as jnp
from jax.experimental import pallas as pl
from jax.experimental.pallas import tpu as pltpu

_NBINS = 38
_IBLK = 64
_JCHUNK = 128
_FLATC = _JCHUNK * _NBINS


def _onehot_kernel(at_ref, bt_ref, aux_ref, w_ref, kflat_ref, out_ref):
    n = bt_ref.shape[2]
    d2 = None
    for c in range(3):
        a = at_ref[0, 0, :, c : c + 1]
        b = bt_ref[0, c : c + 1, :]
        diff = a - b
        d2 = diff * diff if d2 is None else d2 + diff * diff
    d = jnp.sqrt(d2)
    start = aux_ref[1:2, 0:1]
    inv_step = aux_ref[1:2, 1:2]
    bidx = jnp.clip(jnp.floor((d - start) * inv_step), 0.0, _NBINS - 1.0)
    bidx16 = bidx.astype(jnp.bfloat16)
    kflat = kflat_ref[0:1, :]
    for c in range(n // _JCHUNK):
        bc = bidx16[:, c * _JCHUNK : (c + 1) * _JCHUNK]
        m = jnp.dot(bc, w_ref[:, :], preferred_element_type=jnp.float32)
        oh = jnp.where(m == kflat, 1.0, 0.0)
        out_ref[0, :, c * _FLATC : (c + 1) * _FLATC] = oh.astype(jnp.bfloat16)


def kernel(CB_coords, v_bins):
    nbatch, n, _ = CB_coords.shape
    nblk = n // _IBLK
    coords_p = jnp.pad(CB_coords, ((0, 0), (0, 0), (0, 5)))
    coords_rows = coords_p.reshape(nbatch, nblk, _IBLK, 8)
    coords_t = jnp.pad(
        jnp.transpose(CB_coords, (0, 2, 1)), ((0, 0), (0, 5), (0, 0))
    )
    aux = jnp.zeros((8, 128), jnp.float32)
    aux = aux.at[1, 0].set(v_bins[0])
    aux = aux.at[1, 1].set(1.0 / (v_bins[1] - v_bins[0]))
    p = jnp.arange(_FLATC, dtype=jnp.int32)
    w = (p[None, :] // _NBINS == jnp.arange(_JCHUNK, dtype=jnp.int32)[:, None])
    w = w.astype(jnp.bfloat16)
    kflat = (p % _NBINS).astype(jnp.float32)[None, :]

    grid = (nbatch, nblk)
    out = pl.pallas_call(
        _onehot_kernel,
        grid=grid,
        in_specs=[
            pl.BlockSpec((1, 1, _IBLK, 8), lambda b, i: (b, i, 0, 0)),
            pl.BlockSpec((1, 8, n), lambda b, i: (b, 0, 0)),
            pl.BlockSpec((8, 128), lambda b, i: (0, 0)),
            pl.BlockSpec((_JCHUNK, _FLATC), lambda b, i: (0, 0)),
            pl.BlockSpec((1, _FLATC), lambda b, i: (0, 0)),
        ],
        out_specs=pl.BlockSpec((1, _IBLK, n * _NBINS), lambda b, i: (b, i, 0)),
        out_shape=jax.ShapeDtypeStruct((nbatch, n, n * _NBINS), jnp.bfloat16),
    )(coords_rows, coords_t, aux, w, kflat)
    return out.astype(jnp.float32).reshape(nbatch, n, n, _NBINS)
